# baseline (speedup 1.0000x reference)
import functools

import jax
import jax.numpy as jnp
from jax import lax
from jax.experimental import pallas as pl
from jax.experimental.pallas import tpu as pltpu
from jax.experimental.pallas import tpu_sc as plsc

NUM_EXPERTS = 8
TOP_K = 2
BLK = 256
NUM_BLOCKS = (2048 * TOP_K) // BLK + NUM_EXPERTS
PADDED = NUM_BLOCKS * BLK

SC_CORES = 2
SC_SUBCORES = 16
SC_WORKERS = SC_CORES * SC_SUBCORES


def _tc_moe_body(meta_ref, x_ref, w1_ref, w3_ref, w2_ref, wr_ref, out_ref):
    i = pl.program_id(0)

    @pl.when(meta_ref[1, i] == i)
    def _():
        x = x_ref[...]
        a1 = lax.dot_general(x, w1_ref[0], (((1,), (1,)), ((), ())))
        a3 = lax.dot_general(x, w3_ref[0], (((1,), (1,)), ((), ())))
        h = (a1 / (1.0 + jnp.exp(-a1))) * a3
        y = jnp.dot(h, w2_ref[0])
        out_ref[...] = y * wr_ref[0, 0, :][:, None]


def _expert_blocks(x_sorted, meta, w1, w2, w3, w_row3d):
    D = x_sorted.shape[1]
    F = w1.shape[1]
    grid_spec = pltpu.PrefetchScalarGridSpec(
        num_scalar_prefetch=1,
        grid=(NUM_BLOCKS,),
        in_specs=[
            pl.BlockSpec((BLK, D), lambda i, meta: (meta[1, i], 0)),
            pl.BlockSpec((1, F, D), lambda i, meta: (meta[0, i], 0, 0)),
            pl.BlockSpec((1, F, D), lambda i, meta: (meta[0, i], 0, 0)),
            pl.BlockSpec((1, F, D), lambda i, meta: (meta[0, i], 0, 0)),
            pl.BlockSpec((1, 1, BLK), lambda i, meta: (meta[1, i], 0, 0)),
        ],
        out_specs=pl.BlockSpec((BLK, D), lambda i, meta: (meta[1, i], 0)),
    )
    return pl.pallas_call(
        _tc_moe_body,
        grid_spec=grid_spec,
        out_shape=jax.ShapeDtypeStruct((PADDED, D), jnp.float32),
        compiler_params=pltpu.CompilerParams(
            dimension_semantics=("arbitrary",)),
    )(meta, x_sorted, w1, w3, w2, w_row3d)


def _sc_gather(table, idx):
    n, d = idx.shape[0], table.shape[1]
    rpw = n // SC_WORKERS
    ch = rpw
    while ch * d * 4 > 256 * 1024 or ch > 128:
        ch //= 2
    nch = rpw // ch
    mesh = plsc.VectorSubcoreMesh(core_axis_name="c", subcore_axis_name="s")

    @functools.partial(
        pl.kernel,
        mesh=mesh,
        out_type=jax.ShapeDtypeStruct((n, d), jnp.float32),
        scratch_types=[
            pltpu.VMEM((rpw,), jnp.int32),
            pltpu.VMEM((ch, d), jnp.float32),
            pltpu.SemaphoreType.DMA,
        ],
    )
    def gather_k(table_hbm, idx_hbm, out_hbm, idx_v, rows_v, sem):
        wid = lax.axis_index("s") * SC_CORES + lax.axis_index("c")
        base = wid * rpw
        pltpu.sync_copy(idx_hbm.at[pl.ds(base, rpw)], idx_v)

        def chunk(c, carry):
            pltpu.async_copy(
                table_hbm.at[idx_v.at[pl.ds(c * ch, ch)]], rows_v, sem
            ).wait()
            pltpu.sync_copy(rows_v, out_hbm.at[pl.ds(base + c * ch, ch)])
            return carry

        lax.fori_loop(0, nch, chunk, 0)

    return gather_k(table, idx)


def _sc_combine(y_w, p0, p1):
    t, d = p0.shape[0], y_w.shape[1]
    tpw = t // SC_WORKERS
    ch = 32
    nch = tpw // ch
    mesh = plsc.VectorSubcoreMesh(core_axis_name="c", subcore_axis_name="s")

    @functools.partial(
        pl.kernel,
        mesh=mesh,
        out_type=jax.ShapeDtypeStruct((t, d), jnp.float32),
        scratch_types=[
            pltpu.VMEM((tpw,), jnp.int32),
            pltpu.VMEM((tpw,), jnp.int32),
            pltpu.VMEM((ch, d), jnp.float32),
            pltpu.VMEM((ch, d), jnp.float32),
            pltpu.SemaphoreType.DMA,
            pltpu.SemaphoreType.DMA,
        ],
    )
    def combine_k(y_hbm, p0_hbm, p1_hbm, out_hbm, i0_v, i1_v, b0_v, b1_v,
                  s0, s1):
        wid = lax.axis_index("s") * SC_CORES + lax.axis_index("c")
        base = wid * tpw
        pltpu.sync_copy(p0_hbm.at[pl.ds(base, tpw)], i0_v)
        pltpu.sync_copy(p1_hbm.at[pl.ds(base, tpw)], i1_v)

        def chunk(c, carry):
            cp0 = pltpu.async_copy(
                y_hbm.at[i0_v.at[pl.ds(c * ch, ch)]], b0_v, s0)
            cp1 = pltpu.async_copy(
                y_hbm.at[i1_v.at[pl.ds(c * ch, ch)]], b1_v, s1)
            cp0.wait()
            cp1.wait()

            def row(i, carry2):
                def col(j, carry3):
                    b0_v[i, pl.ds(j * 16, 16)] = (
                        b0_v[i, pl.ds(j * 16, 16)]
                        + b1_v[i, pl.ds(j * 16, 16)])
                    return carry3

                return lax.fori_loop(0, d // 16, col, carry2)

            lax.fori_loop(0, ch, row, 0)
            pltpu.sync_copy(b0_v, out_hbm.at[pl.ds(base + c * ch, ch)])
            return carry

        lax.fori_loop(0, nch, chunk, 0)

    return combine_k(y_w, p0, p1)


def kernel(inputs, gate_w, w1, w2, w3):
    T, D = inputs.shape
    E = gate_w.shape[0]
    A = T * TOP_K

    gate_logits = inputs @ gate_w.T
    weights, selected = jax.lax.top_k(gate_logits, TOP_K)
    weights = jax.nn.softmax(weights.astype(jnp.float32), axis=1)
    weights = weights.astype(inputs.dtype)

    e_flat = selected.reshape(-1).astype(jnp.int32)
    w_flat = weights.reshape(-1)
    onehot = (e_flat[:, None] == jnp.arange(E, dtype=jnp.int32)[None, :])
    onehot = onehot.astype(jnp.int32)
    counts = jnp.sum(onehot, axis=0)
    within = jnp.cumsum(onehot, axis=0) - 1
    within = jnp.take_along_axis(within, e_flat[:, None], axis=1)[:, 0]
    blocks_e = (counts + BLK - 1) // BLK
    blk_cum = jnp.cumsum(blocks_e)
    row_start_e = (blk_cum - blocks_e) * BLK
    pos = row_start_e[e_flat] + within
    nb = blk_cum[-1]

    token_src = jnp.zeros((PADDED,), jnp.int32).at[pos].set(
        jnp.arange(A, dtype=jnp.int32) // TOP_K)
    w_row = jnp.zeros((PADDED,), jnp.float32).at[pos].set(w_flat)
    w_row3d = w_row.reshape(NUM_BLOCKS, 1, BLK)

    bidx = jnp.arange(NUM_BLOCKS, dtype=jnp.int32)
    bexp = jnp.searchsorted(blk_cum, bidx, side="right").astype(jnp.int32)
    bexp = jnp.minimum(bexp, E - 1)
    bxi = jnp.minimum(bidx, nb - 1)
    meta = jnp.stack([bexp, bxi])

    p01 = pos.reshape(T, TOP_K)

    x_sorted = _sc_gather(inputs, token_src)
    y_w = _expert_blocks(x_sorted, meta, w1, w2, w3, w_row3d)
    out = _sc_combine(y_w, p01[:, 0].astype(jnp.int32),
                      p01[:, 1].astype(jnp.int32))
    return out


# speedup vs baseline: 1.0240x; 1.0240x over previous
import functools

import jax
import jax.numpy as jnp
from jax import lax
from jax.experimental import pallas as pl
from jax.experimental.pallas import tpu as pltpu
from jax.experimental.pallas import tpu_sc as plsc

NUM_EXPERTS = 8
TOP_K = 2
BLK = 256
NUM_BLOCKS = (2048 * TOP_K) // BLK + NUM_EXPERTS
PADDED = NUM_BLOCKS * BLK

SC_CORES = 2
SC_SUBCORES = 16
SC_WORKERS = SC_CORES * SC_SUBCORES


def _tc_moe_body(meta_ref, x_ref, w1_ref, w3_ref, w2_ref, wr_ref, out_ref):
    i = pl.program_id(0)

    @pl.when(meta_ref[1, i] == i)
    def _():
        x = x_ref[...]
        a1 = lax.dot_general(x, w1_ref[0], (((1,), (1,)), ((), ())))
        a3 = lax.dot_general(x, w3_ref[0], (((1,), (1,)), ((), ())))
        h = (a1 / (1.0 + jnp.exp(-a1))) * a3
        y = jnp.dot(h, w2_ref[0])
        out_ref[...] = y * wr_ref[0, 0, :][:, None]


def _expert_blocks(x_sorted, meta, w1, w2, w3, w_row3d):
    D = x_sorted.shape[1]
    F = w1.shape[1]
    grid_spec = pltpu.PrefetchScalarGridSpec(
        num_scalar_prefetch=1,
        grid=(NUM_BLOCKS,),
        in_specs=[
            pl.BlockSpec((BLK, D), lambda i, meta: (meta[1, i], 0)),
            pl.BlockSpec((1, F, D), lambda i, meta: (meta[0, i], 0, 0)),
            pl.BlockSpec((1, F, D), lambda i, meta: (meta[0, i], 0, 0)),
            pl.BlockSpec((1, F, D), lambda i, meta: (meta[0, i], 0, 0)),
            pl.BlockSpec((1, 1, BLK), lambda i, meta: (meta[1, i], 0, 0)),
        ],
        out_specs=pl.BlockSpec((BLK, D), lambda i, meta: (meta[1, i], 0)),
    )
    return pl.pallas_call(
        _tc_moe_body,
        grid_spec=grid_spec,
        out_shape=jax.ShapeDtypeStruct((PADDED, D), jnp.float32),
        compiler_params=pltpu.CompilerParams(
            dimension_semantics=("arbitrary",)),
    )(meta, x_sorted, w1, w3, w2, w_row3d)


def _sc_gather(table, idx, ch):
    n, d = idx.shape[0], table.shape[1]
    rpw = n // SC_WORKERS
    nch = rpw // ch
    mesh = plsc.VectorSubcoreMesh(core_axis_name="c", subcore_axis_name="s")

    @functools.partial(
        pl.kernel,
        mesh=mesh,
        out_type=jax.ShapeDtypeStruct((n, d), jnp.float32),
        scratch_types=[
            pltpu.VMEM((rpw,), jnp.int32),
            pltpu.VMEM((ch, d), jnp.float32),
            pltpu.VMEM((ch, d), jnp.float32),
            pltpu.SemaphoreType.DMA,
            pltpu.SemaphoreType.DMA,
            pltpu.SemaphoreType.DMA,
            pltpu.SemaphoreType.DMA,
        ],
    )
    def gather_k(table_hbm, idx_hbm, out_hbm, idx_v, buf0, buf1,
                 g0, g1, s0, s1):
        wid = lax.axis_index("s") * SC_CORES + lax.axis_index("c")
        base = wid * rpw
        pltpu.sync_copy(idx_hbm.at[pl.ds(base, rpw)], idx_v)
        bufs, gsem, ssem = [buf0, buf1], [g0, g1], [s0, s1]
        gat = [None, None]
        sto = [None, None]
        gat[0] = pltpu.async_copy(
            table_hbm.at[idx_v.at[pl.ds(0, ch)]], bufs[0], gsem[0])
        for i in range(nch):
            b = i % 2
            nb_ = (i + 1) % 2
            if i + 1 < nch:
                if sto[nb_] is not None:
                    sto[nb_].wait()
                gat[nb_] = pltpu.async_copy(
                    table_hbm.at[idx_v.at[pl.ds((i + 1) * ch, ch)]],
                    bufs[nb_], gsem[nb_])
            gat[b].wait()
            sto[b] = pltpu.async_copy(
                bufs[b], out_hbm.at[pl.ds(base + i * ch, ch)], ssem[b])
        for s in sto:
            if s is not None:
                s.wait()

    return gather_k(table, idx)


def _tc_add_body(a_ref, b_ref, out_ref):
    out_ref[...] = a_ref[...] + b_ref[...]


def _tc_pair_add(g, t, blk):
    d = g.shape[1]
    nblk = t // blk
    return pl.pallas_call(
        _tc_add_body,
        grid=(nblk,),
        in_specs=[
            pl.BlockSpec((blk, d), lambda i: (i, 0)),
            pl.BlockSpec((blk, d), lambda i: (i + nblk, 0)),
        ],
        out_specs=pl.BlockSpec((blk, d), lambda i: (i, 0)),
        out_shape=jax.ShapeDtypeStruct((t, d), jnp.float32),
    )(g, g)


def kernel(inputs, gate_w, w1, w2, w3):
    T, D = inputs.shape
    E = gate_w.shape[0]
    A = T * TOP_K

    gate_logits = inputs @ gate_w.T
    weights, selected = jax.lax.top_k(gate_logits, TOP_K)
    weights = jax.nn.softmax(weights.astype(jnp.float32), axis=1)
    weights = weights.astype(inputs.dtype)

    e_flat = selected.reshape(-1).astype(jnp.int32)
    w_flat = weights.reshape(-1)
    onehot = (e_flat[:, None] == jnp.arange(E, dtype=jnp.int32)[None, :])
    onehot = onehot.astype(jnp.int32)
    counts = jnp.sum(onehot, axis=0)
    within = jnp.cumsum(onehot, axis=0) - 1
    within = jnp.take_along_axis(within, e_flat[:, None], axis=1)[:, 0]
    blocks_e = (counts + BLK - 1) // BLK
    blk_cum = jnp.cumsum(blocks_e)
    row_start_e = (blk_cum - blocks_e) * BLK
    pos = row_start_e[e_flat] + within
    nb = blk_cum[-1]

    token_src = jnp.zeros((PADDED,), jnp.int32).at[pos].set(
        jnp.arange(A, dtype=jnp.int32) // TOP_K)
    w_row = jnp.zeros((PADDED,), jnp.float32).at[pos].set(w_flat)
    w_row3d = w_row.reshape(NUM_BLOCKS, 1, BLK)

    bidx = jnp.arange(NUM_BLOCKS, dtype=jnp.int32)
    bexp = jnp.searchsorted(blk_cum, bidx, side="right").astype(jnp.int32)
    bexp = jnp.minimum(bexp, E - 1)
    bxi = jnp.minimum(bidx, nb - 1)
    meta = jnp.stack([bexp, bxi])

    p01 = pos.reshape(T, TOP_K)
    p_cat = jnp.concatenate([p01[:, 0], p01[:, 1]]).astype(jnp.int32)

    x_sorted = _sc_gather(inputs, token_src, ch=48)
    y_w = _expert_blocks(x_sorted, meta, w1, w2, w3, w_row3d)
    g = _sc_gather(y_w, p_cat, ch=32)
    return _tc_pair_add(g, T, blk=512)


# speedup vs baseline: 1.0290x; 1.0049x over previous
import functools

import jax
import jax.numpy as jnp
from jax import lax
from jax.experimental import pallas as pl
from jax.experimental.pallas import tpu as pltpu
from jax.experimental.pallas import tpu_sc as plsc

NUM_EXPERTS = 8
TOP_K = 2
BLK = 256
NUM_BLOCKS = (2048 * TOP_K) // BLK + NUM_EXPERTS
PADDED = NUM_BLOCKS * BLK

SC_CORES = 2
SC_SUBCORES = 16
SC_WORKERS = SC_CORES * SC_SUBCORES


def _tc_moe_body(meta_ref, x_ref, w1_ref, w3_ref, w2_ref, wr_ref, out_ref):
    i = pl.program_id(0)

    @pl.when(meta_ref[1, i] == i)
    def _():
        x = x_ref[...]
        a1 = lax.dot_general(x, w1_ref[0], (((1,), (1,)), ((), ())))
        a3 = lax.dot_general(x, w3_ref[0], (((1,), (1,)), ((), ())))
        h = (a1 / (1.0 + jnp.exp(-a1))) * a3
        y = jnp.dot(h, w2_ref[0])
        out_ref[...] = y * wr_ref[0, 0, :][:, None]


def _expert_blocks(x_sorted, meta, w1, w2, w3, w_row3d):
    D = x_sorted.shape[1]
    F = w1.shape[1]
    grid_spec = pltpu.PrefetchScalarGridSpec(
        num_scalar_prefetch=1,
        grid=(NUM_BLOCKS,),
        in_specs=[
            pl.BlockSpec((BLK, D), lambda i, meta: (meta[1, i], 0)),
            pl.BlockSpec((1, F, D), lambda i, meta: (meta[0, i], 0, 0)),
            pl.BlockSpec((1, F, D), lambda i, meta: (meta[0, i], 0, 0)),
            pl.BlockSpec((1, F, D), lambda i, meta: (meta[0, i], 0, 0)),
            pl.BlockSpec((1, 1, BLK), lambda i, meta: (meta[1, i], 0, 0)),
        ],
        out_specs=pl.BlockSpec((BLK, D), lambda i, meta: (meta[1, i], 0)),
    )
    return pl.pallas_call(
        _tc_moe_body,
        grid_spec=grid_spec,
        out_shape=jax.ShapeDtypeStruct((PADDED, D), jnp.float32),
        compiler_params=pltpu.CompilerParams(
            dimension_semantics=("arbitrary",)),
        name="tc_expert_blocks",
    )(meta, x_sorted, w1, w3, w2, w_row3d)


def _sc_gather(table, idx, ch, name):
    n, d = idx.shape[0], table.shape[1]
    rpw = n // SC_WORKERS
    nch = rpw // ch
    mesh = plsc.VectorSubcoreMesh(core_axis_name="c", subcore_axis_name="s")

    @functools.partial(
        pl.kernel,
        mesh=mesh,
        name=name,
        out_type=jax.ShapeDtypeStruct((n, d), jnp.float32),
        scratch_types=[
            pltpu.VMEM((rpw,), jnp.int32),
            pltpu.VMEM((ch, d), jnp.float32),
            pltpu.VMEM((ch, d), jnp.float32),
            pltpu.SemaphoreType.DMA,
            pltpu.SemaphoreType.DMA,
            pltpu.SemaphoreType.DMA,
            pltpu.SemaphoreType.DMA,
        ],
    )
    def gather_k(table_hbm, idx_hbm, out_hbm, idx_v, buf0, buf1,
                 g0, g1, s0, s1):
        wid = lax.axis_index("s") * SC_CORES + lax.axis_index("c")
        base = wid * rpw
        pltpu.sync_copy(idx_hbm.at[pl.ds(base, rpw)], idx_v)
        bufs, gsem, ssem = [buf0, buf1], [g0, g1], [s0, s1]
        gat = [None, None]
        sto = [None, None]
        gat[0] = pltpu.async_copy(
            table_hbm.at[idx_v.at[pl.ds(0, ch)]], bufs[0], gsem[0])
        for i in range(nch):
            b = i % 2
            nb_ = (i + 1) % 2
            if i + 1 < nch:
                if sto[nb_] is not None:
                    sto[nb_].wait()
                gat[nb_] = pltpu.async_copy(
                    table_hbm.at[idx_v.at[pl.ds((i + 1) * ch, ch)]],
                    bufs[nb_], gsem[nb_])
            gat[b].wait()
            sto[b] = pltpu.async_copy(
                bufs[b], out_hbm.at[pl.ds(base + i * ch, ch)], ssem[b])
        for s in sto:
            if s is not None:
                s.wait()

    return gather_k(table, idx)


def _tc_add_body(a_ref, b_ref, out_ref):
    out_ref[...] = a_ref[...] + b_ref[...]


def _tc_pair_add(g, t, blk):
    d = g.shape[1]
    nblk = t // blk
    return pl.pallas_call(
        _tc_add_body,
        grid=(nblk,),
        in_specs=[
            pl.BlockSpec((blk, d), lambda i: (i, 0)),
            pl.BlockSpec((blk, d), lambda i: (i + nblk, 0)),
        ],
        out_specs=pl.BlockSpec((blk, d), lambda i: (i, 0)),
        out_shape=jax.ShapeDtypeStruct((t, d), jnp.float32),
        name="tc_pair_add",
    )(g, g)


def kernel(inputs, gate_w, w1, w2, w3):
    T, D = inputs.shape
    E = gate_w.shape[0]
    A = T * TOP_K

    gate_logits = inputs @ gate_w.T
    weights, selected = jax.lax.top_k(gate_logits, TOP_K)
    weights = jax.nn.softmax(weights.astype(jnp.float32), axis=1)
    weights = weights.astype(inputs.dtype)

    e_flat = selected.reshape(-1).astype(jnp.int32)
    w_flat = weights.reshape(-1)
    onehot = (e_flat[:, None] == jnp.arange(E, dtype=jnp.int32)[None, :])
    onehot = onehot.astype(jnp.int32)
    counts = jnp.sum(onehot, axis=0)
    within = jnp.cumsum(onehot, axis=0) - 1
    within = jnp.take_along_axis(within, e_flat[:, None], axis=1)[:, 0]
    blocks_e = (counts + BLK - 1) // BLK
    blk_cum = jnp.cumsum(blocks_e)
    row_start_e = (blk_cum - blocks_e) * BLK
    pos = row_start_e[e_flat] + within
    nb = blk_cum[-1]

    token_src = jnp.zeros((PADDED,), jnp.int32).at[pos].set(
        jnp.arange(A, dtype=jnp.int32) // TOP_K)
    w_row = jnp.zeros((PADDED,), jnp.float32).at[pos].set(w_flat)
    w_row3d = w_row.reshape(NUM_BLOCKS, 1, BLK)

    bidx = jnp.arange(NUM_BLOCKS, dtype=jnp.int32)
    bexp = jnp.searchsorted(blk_cum, bidx, side="right").astype(jnp.int32)
    bexp = jnp.minimum(bexp, E - 1)
    bxi = jnp.minimum(bidx, nb - 1)
    meta = jnp.stack([bexp, bxi])

    p01 = pos.reshape(T, TOP_K)
    p_cat = jnp.concatenate([p01[:, 0], p01[:, 1]]).astype(jnp.int32)

    x_sorted = _sc_gather(inputs, token_src, ch=48, name="sc_gather_x")
    y_w = _expert_blocks(x_sorted, meta, w1, w2, w3, w_row3d)
    g = _sc_gather(y_w, p_cat, ch=32, name="sc_gather_y")
    return _tc_pair_add(g, T, blk=512)


# speedup vs baseline: 1.6136x; 1.5681x over previous
import functools

import jax
import jax.numpy as jnp
from jax import lax
from jax.experimental import pallas as pl
from jax.experimental.pallas import tpu as pltpu
from jax.experimental.pallas import tpu_sc as plsc

NUM_EXPERTS = 8
TOP_K = 2
BLK = 256
NUM_BLOCKS = (2048 * TOP_K) // BLK + NUM_EXPERTS
PADDED = NUM_BLOCKS * BLK

SC_CORES = 2
SC_SUBCORES = 16
SC_WORKERS = SC_CORES * SC_SUBCORES


def _tc_moe_body(meta_ref, x_ref, w1_ref, w3_ref, w2_ref, wr_ref, out_ref):
    i = pl.program_id(0)

    @pl.when(meta_ref[1, i] == i)
    def _():
        x = x_ref[...]
        a1 = lax.dot_general(x, w1_ref[0], (((1,), (1,)), ((), ())))
        a3 = lax.dot_general(x, w3_ref[0], (((1,), (1,)), ((), ())))
        h = (a1 / (1.0 + jnp.exp(-a1))) * a3
        y = jnp.dot(h, w2_ref[0])
        out_ref[...] = y * wr_ref[0, 0, :][:, None]


def _expert_blocks(x_sorted, meta, w1, w2, w3, w_row3d):
    D = x_sorted.shape[1]
    F = w1.shape[1]
    grid_spec = pltpu.PrefetchScalarGridSpec(
        num_scalar_prefetch=1,
        grid=(NUM_BLOCKS,),
        in_specs=[
            pl.BlockSpec((BLK, D), lambda i, meta: (meta[1, i], 0)),
            pl.BlockSpec((1, F, D), lambda i, meta: (meta[0, i], 0, 0)),
            pl.BlockSpec((1, F, D), lambda i, meta: (meta[0, i], 0, 0)),
            pl.BlockSpec((1, F, D), lambda i, meta: (meta[0, i], 0, 0)),
            pl.BlockSpec((1, 1, BLK), lambda i, meta: (meta[1, i], 0, 0)),
        ],
        out_specs=pl.BlockSpec((BLK, D), lambda i, meta: (meta[1, i], 0)),
    )
    return pl.pallas_call(
        _tc_moe_body,
        grid_spec=grid_spec,
        out_shape=jax.ShapeDtypeStruct((PADDED, D), jnp.float32),
        compiler_params=pltpu.CompilerParams(
            dimension_semantics=("arbitrary",)),
        name="tc_expert_blocks",
    )(meta, x_sorted, w1, w3, w2, w_row3d)


def _sc_gather(table, idx, ch, name):
    n, d = idx.shape[0], table.shape[1]
    rpw = n // SC_WORKERS
    nch = rpw // ch
    mesh = plsc.VectorSubcoreMesh(core_axis_name="c", subcore_axis_name="s")

    @functools.partial(
        pl.kernel,
        mesh=mesh,
        name=name,
        out_type=jax.ShapeDtypeStruct((n, d), jnp.float32),
        scratch_types=[
            pltpu.VMEM((rpw,), jnp.int32),
            pltpu.VMEM((ch, d), jnp.float32),
            pltpu.VMEM((ch, d), jnp.float32),
            pltpu.SemaphoreType.DMA,
            pltpu.SemaphoreType.DMA,
            pltpu.SemaphoreType.DMA,
            pltpu.SemaphoreType.DMA,
        ],
    )
    def gather_k(table_hbm, idx_hbm, out_hbm, idx_v, buf0, buf1,
                 g0, g1, s0, s1):
        wid = lax.axis_index("s") * SC_CORES + lax.axis_index("c")
        base = wid * rpw
        pltpu.sync_copy(idx_hbm.at[pl.ds(base, rpw)], idx_v)
        bufs, gsem, ssem = [buf0, buf1], [g0, g1], [s0, s1]
        gat = [None, None]
        sto = [None, None]
        gat[0] = pltpu.async_copy(
            table_hbm.at[idx_v.at[pl.ds(0, ch)]], bufs[0], gsem[0])
        for i in range(nch):
            b = i % 2
            nb_ = (i + 1) % 2
            if i + 1 < nch:
                if sto[nb_] is not None:
                    sto[nb_].wait()
                gat[nb_] = pltpu.async_copy(
                    table_hbm.at[idx_v.at[pl.ds((i + 1) * ch, ch)]],
                    bufs[nb_], gsem[nb_])
            gat[b].wait()
            sto[b] = pltpu.async_copy(
                bufs[b], out_hbm.at[pl.ds(base + i * ch, ch)], ssem[b])
        for s in sto:
            if s is not None:
                s.wait()

    return gather_k(table, idx)


def _tc_add_body(a_ref, b_ref, out_ref):
    out_ref[...] = a_ref[...] + b_ref[...]


def _tc_pair_add(g, t, blk):
    d = g.shape[1]
    nblk = t // blk
    return pl.pallas_call(
        _tc_add_body,
        grid=(nblk,),
        in_specs=[
            pl.BlockSpec((blk, d), lambda i: (i, 0)),
            pl.BlockSpec((blk, d), lambda i: (i + nblk, 0)),
        ],
        out_specs=pl.BlockSpec((blk, d), lambda i: (i, 0)),
        out_shape=jax.ShapeDtypeStruct((t, d), jnp.float32),
        name="tc_pair_add",
    )(g, g)


def kernel(inputs, gate_w, w1, w2, w3):
    T, D = inputs.shape
    E = gate_w.shape[0]
    A = T * TOP_K

    gate_logits = inputs @ gate_w.T
    weights, selected = jax.lax.top_k(gate_logits, TOP_K)
    weights = jax.nn.softmax(weights.astype(jnp.float32), axis=1)
    weights = weights.astype(inputs.dtype)

    e_flat = selected.reshape(-1).astype(jnp.int32)
    w_flat = weights.reshape(-1)
    onehot = (e_flat[:, None] == jnp.arange(E, dtype=jnp.int32)[None, :])
    onehot = onehot.astype(jnp.int32)
    counts = jnp.sum(onehot, axis=0)
    within = jnp.cumsum(onehot, axis=0) - 1
    within = jnp.take_along_axis(within, e_flat[:, None], axis=1)[:, 0]
    blocks_e = (counts + BLK - 1) // BLK
    blk_cum = jnp.cumsum(blocks_e)
    row_start_e = (blk_cum - blocks_e) * BLK
    pos = row_start_e[e_flat] + within
    nb = blk_cum[-1]

    token_src = (jnp.arange(PADDED, dtype=jnp.int32) % T).at[pos].set(
        jnp.arange(A, dtype=jnp.int32) // TOP_K)
    w_row = jnp.zeros((PADDED,), jnp.float32).at[pos].set(w_flat)
    w_row3d = w_row.reshape(NUM_BLOCKS, 1, BLK)

    bidx = jnp.arange(NUM_BLOCKS, dtype=jnp.int32)
    bexp = jnp.searchsorted(blk_cum, bidx, side="right").astype(jnp.int32)
    bexp = jnp.minimum(bexp, E - 1)
    bxi = jnp.minimum(bidx, nb - 1)
    meta = jnp.stack([bexp, bxi])

    p01 = pos.reshape(T, TOP_K)
    p_cat = jnp.concatenate([p01[:, 0], p01[:, 1]]).astype(jnp.int32)

    x_sorted = _sc_gather(inputs, token_src, ch=48, name="sc_gather_x")
    y_w = _expert_blocks(x_sorted, meta, w1, w2, w3, w_row3d)
    g = _sc_gather(y_w, p_cat, ch=32, name="sc_gather_y")
    return _tc_pair_add(g, T, blk=512)


# speedup vs baseline: 1.7334x; 1.0743x over previous
import functools

import jax
import jax.numpy as jnp
from jax import lax
from jax.experimental import pallas as pl
from jax.experimental.pallas import tpu as pltpu
from jax.experimental.pallas import tpu_sc as plsc

NUM_EXPERTS = 8
TOP_K = 2
BLK = 256
NUM_BLOCKS = (2048 * TOP_K) // BLK + NUM_EXPERTS
PADDED = NUM_BLOCKS * BLK

SC_CORES = 2
SC_SUBCORES = 16
SC_WORKERS = SC_CORES * SC_SUBCORES


def _tc_moe_body(meta_ref, x_ref, w1_ref, w3_ref, w2_ref, wr_ref, out_ref):
    i = pl.program_id(0)

    @pl.when(meta_ref[1, i] == i)
    def _():
        x = x_ref[...]
        a1 = lax.dot_general(x, w1_ref[0], (((1,), (1,)), ((), ())))
        a3 = lax.dot_general(x, w3_ref[0], (((1,), (1,)), ((), ())))
        h = (a1 / (1.0 + jnp.exp(-a1))) * a3
        y = jnp.dot(h, w2_ref[0])
        out_ref[...] = y * wr_ref[0, 0, :][:, None]


def _expert_blocks(x_sorted, meta, w1, w2, w3, w_row3d):
    D = x_sorted.shape[1]
    F = w1.shape[1]
    grid_spec = pltpu.PrefetchScalarGridSpec(
        num_scalar_prefetch=1,
        grid=(NUM_BLOCKS,),
        in_specs=[
            pl.BlockSpec((BLK, D), lambda i, meta: (meta[1, i], 0)),
            pl.BlockSpec((1, F, D), lambda i, meta: (meta[0, i], 0, 0)),
            pl.BlockSpec((1, F, D), lambda i, meta: (meta[0, i], 0, 0)),
            pl.BlockSpec((1, F, D), lambda i, meta: (meta[0, i], 0, 0)),
            pl.BlockSpec((1, 1, BLK), lambda i, meta: (meta[1, i], 0, 0)),
        ],
        out_specs=pl.BlockSpec((BLK, D), lambda i, meta: (meta[1, i], 0)),
    )
    return pl.pallas_call(
        _tc_moe_body,
        grid_spec=grid_spec,
        out_shape=jax.ShapeDtypeStruct((PADDED, D), jnp.float32),
        compiler_params=pltpu.CompilerParams(
            dimension_semantics=("arbitrary",)),
        name="tc_expert_blocks",
    )(meta, x_sorted, w1, w3, w2, w_row3d)


def _sc_scatter_gather_x(table, pos, tok, wts, n, ch):
    a = pos.shape[0]
    t_tokens, d = table.shape
    rpw = n // SC_WORKERS
    nch = rpw // ch
    mesh = plsc.VectorSubcoreMesh(core_axis_name="c", subcore_axis_name="s")

    @functools.partial(
        pl.kernel,
        mesh=mesh,
        name="sc_gather_x",
        compiler_params=pltpu.CompilerParams(needs_layout_passes=False),
        out_type=(
            jax.ShapeDtypeStruct((n, d), jnp.float32),
            jax.ShapeDtypeStruct((n,), jnp.float32),
        ),
        scratch_types=[
            pltpu.VMEM((a,), jnp.int32),
            pltpu.VMEM((a,), jnp.int32),
            pltpu.VMEM((a,), jnp.float32),
            pltpu.VMEM((n,), jnp.int32),
            pltpu.VMEM((n,), jnp.float32),
            pltpu.VMEM((ch, d), jnp.float32),
            pltpu.VMEM((ch, d), jnp.float32),
            pltpu.SemaphoreType.DMA,
            pltpu.SemaphoreType.DMA,
            pltpu.SemaphoreType.DMA,
            pltpu.SemaphoreType.DMA,
        ],
    )
    def gather_k(table_hbm, pos_hbm, tok_hbm, w_hbm, out_hbm, wrow_hbm,
                 pos_v, tok_v, w_v, ts_v, wr_v, buf0, buf1, g0, g1, s0, s1):
        wid = lax.axis_index("s") * SC_CORES + lax.axis_index("c")
        base = wid * rpw
        pltpu.sync_copy(pos_hbm.at[:], pos_v)
        pltpu.sync_copy(tok_hbm.at[:], tok_v)
        pltpu.sync_copy(w_hbm.at[:], w_v)

        lanes = lax.iota(jnp.int32, 16)
        zeros16 = lanes.astype(jnp.float32) * 0.0

        def init(i, carry):
            v = (i * 16 + lanes) & (t_tokens - 1)
            ts_v[pl.ds(i * 16, 16)] = v
            wr_v[pl.ds(i * 16, 16)] = zeros16
            return carry

        lax.fori_loop(0, n // 16, init, 0)

        def scat(i, carry):
            p = pos_v[pl.ds(i * 16, 16)]
            plsc.store_scatter(ts_v, [p], tok_v[pl.ds(i * 16, 16)])
            plsc.store_scatter(wr_v, [p], w_v[pl.ds(i * 16, 16)])
            return carry

        lax.fori_loop(0, a // 16, scat, 0)

        @pl.when(wid == 0)
        def _():
            pltpu.sync_copy(wr_v, wrow_hbm.at[:])

        bufs, gsem, ssem = [buf0, buf1], [g0, g1], [s0, s1]
        gat = [None, None]
        sto = [None, None]
        gat[0] = pltpu.async_copy(
            table_hbm.at[ts_v.at[pl.ds(base, ch)]], bufs[0], gsem[0])
        for i in range(nch):
            b = i % 2
            nb_ = (i + 1) % 2
            if i + 1 < nch:
                if sto[nb_] is not None:
                    sto[nb_].wait()
                gat[nb_] = pltpu.async_copy(
                    table_hbm.at[ts_v.at[pl.ds(base + (i + 1) * ch, ch)]],
                    bufs[nb_], gsem[nb_])
            gat[b].wait()
            sto[b] = pltpu.async_copy(
                bufs[b], out_hbm.at[pl.ds(base + i * ch, ch)], ssem[b])
        for s in sto:
            if s is not None:
                s.wait()

    return gather_k(table, pos, tok, wts)


def _sc_gather(table, idx, ch, name):
    n, d = idx.shape[0], table.shape[1]
    rpw = n // SC_WORKERS
    nch = rpw // ch
    mesh = plsc.VectorSubcoreMesh(core_axis_name="c", subcore_axis_name="s")

    @functools.partial(
        pl.kernel,
        mesh=mesh,
        name=name,
        out_type=jax.ShapeDtypeStruct((n, d), jnp.float32),
        scratch_types=[
            pltpu.VMEM((rpw,), jnp.int32),
            pltpu.VMEM((ch, d), jnp.float32),
            pltpu.VMEM((ch, d), jnp.float32),
            pltpu.SemaphoreType.DMA,
            pltpu.SemaphoreType.DMA,
            pltpu.SemaphoreType.DMA,
            pltpu.SemaphoreType.DMA,
        ],
    )
    def gather_k(table_hbm, idx_hbm, out_hbm, idx_v, buf0, buf1,
                 g0, g1, s0, s1):
        wid = lax.axis_index("s") * SC_CORES + lax.axis_index("c")
        base = wid * rpw
        pltpu.sync_copy(idx_hbm.at[pl.ds(base, rpw)], idx_v)
        bufs, gsem, ssem = [buf0, buf1], [g0, g1], [s0, s1]
        gat = [None, None]
        sto = [None, None]
        gat[0] = pltpu.async_copy(
            table_hbm.at[idx_v.at[pl.ds(0, ch)]], bufs[0], gsem[0])
        for i in range(nch):
            b = i % 2
            nb_ = (i + 1) % 2
            if i + 1 < nch:
                if sto[nb_] is not None:
                    sto[nb_].wait()
                gat[nb_] = pltpu.async_copy(
                    table_hbm.at[idx_v.at[pl.ds((i + 1) * ch, ch)]],
                    bufs[nb_], gsem[nb_])
            gat[b].wait()
            sto[b] = pltpu.async_copy(
                bufs[b], out_hbm.at[pl.ds(base + i * ch, ch)], ssem[b])
        for s in sto:
            if s is not None:
                s.wait()

    return gather_k(table, idx)


def _tc_add_body(a_ref, b_ref, out_ref):
    out_ref[...] = a_ref[...] + b_ref[...]


def _tc_pair_add(g, t, blk):
    d = g.shape[1]
    nblk = t // blk
    return pl.pallas_call(
        _tc_add_body,
        grid=(nblk,),
        in_specs=[
            pl.BlockSpec((blk, d), lambda i: (i, 0)),
            pl.BlockSpec((blk, d), lambda i: (i + nblk, 0)),
        ],
        out_specs=pl.BlockSpec((blk, d), lambda i: (i, 0)),
        out_shape=jax.ShapeDtypeStruct((t, d), jnp.float32),
        name="tc_pair_add",
    )(g, g)


def kernel(inputs, gate_w, w1, w2, w3):
    T, D = inputs.shape
    E = gate_w.shape[0]
    A = T * TOP_K

    gate_logits = inputs @ gate_w.T
    weights, selected = jax.lax.top_k(gate_logits, TOP_K)
    weights = jax.nn.softmax(weights.astype(jnp.float32), axis=1)
    weights = weights.astype(inputs.dtype)

    e_flat = selected.reshape(-1).astype(jnp.int32)
    w_flat = weights.reshape(-1)
    onehot = (e_flat[:, None] == jnp.arange(E, dtype=jnp.int32)[None, :])
    onehot = onehot.astype(jnp.int32)
    counts = jnp.sum(onehot, axis=0)
    within = jnp.cumsum(onehot, axis=0) - 1
    within = jnp.take_along_axis(within, e_flat[:, None], axis=1)[:, 0]
    blocks_e = (counts + BLK - 1) // BLK
    blk_cum = jnp.cumsum(blocks_e)
    row_start_e = (blk_cum - blocks_e) * BLK
    pos = row_start_e[e_flat] + within
    nb = blk_cum[-1]

    tok = jnp.arange(A, dtype=jnp.int32) // TOP_K

    bidx = jnp.arange(NUM_BLOCKS, dtype=jnp.int32)
    bexp = jnp.searchsorted(blk_cum, bidx, side="right").astype(jnp.int32)
    bexp = jnp.minimum(bexp, E - 1)
    bxi = jnp.minimum(bidx, nb - 1)
    meta = jnp.stack([bexp, bxi])

    p01 = pos.reshape(T, TOP_K)
    p_cat = jnp.concatenate([p01[:, 0], p01[:, 1]]).astype(jnp.int32)

    x_sorted, w_row = _sc_scatter_gather_x(
        inputs, pos.astype(jnp.int32), tok, w_flat, PADDED, ch=48)
    w_row3d = w_row.reshape(NUM_BLOCKS, 1, BLK)
    y_w = _expert_blocks(x_sorted, meta, w1, w2, w3, w_row3d)
    g = _sc_gather(y_w, p_cat, ch=32, name="sc_gather_y")
    return _tc_pair_add(g, T, blk=512)
